# baseline (speedup 1.0000x reference)
import functools

import jax
import jax.numpy as jnp
from jax.experimental import pallas as pl
from jax.experimental.pallas import tpu as pltpu

D_MODEL = 1024
D_FF = 4096
NUM_EXPERTS = 8

_TR = 1024
_T = 512
_FB = 512


def _router_body(x_ref, wr_ref, e1_ref, e2_ref, mask_ref):
    x = x_ref[...]
    logits = jax.lax.dot_general(
        x, wr_ref[...], (((1,), (1,)), ((), ())),
        preferred_element_type=jnp.float32)
    col = jax.lax.broadcasted_iota(jnp.int32, logits.shape, 1)
    neg = jnp.float32(-1e30)
    logits = jnp.where(col < NUM_EXPERTS, logits, neg)
    m1 = jnp.max(logits, axis=1, keepdims=True)
    e1 = jnp.min(jnp.where(logits == m1, col, 127), axis=1)
    l2 = jnp.where(col == e1[:, None], neg, logits)
    m2 = jnp.max(l2, axis=1, keepdims=True)
    e2 = jnp.min(jnp.where(l2 == m2, col, 127), axis=1)
    e1_ref[...] = e1.reshape(1, 1, _TR)
    e2_ref[...] = e2.reshape(1, 1, _TR)
    mask_ref[...] = jnp.where(
        (col == e1[:, None]) | (col == e2[:, None]), 1.0, 0.0)


def _route(x2d, wr_pad):
    n = x2d.shape[0]
    g = n // _TR
    espec = pl.BlockSpec((1, 1, _TR), lambda i: (i, 0, 0))
    return pl.pallas_call(
        _router_body,
        grid=(g,),
        in_specs=[
            pl.BlockSpec((_TR, D_MODEL), lambda i: (i, 0)),
            pl.BlockSpec((128, D_MODEL), lambda i: (0, 0)),
        ],
        out_specs=[espec, espec,
                   pl.BlockSpec((_TR, 128), lambda i: (i, 0))],
        out_shape=[
            jax.ShapeDtypeStruct((g, 1, _TR), jnp.int32),
            jax.ShapeDtypeStruct((g, 1, _TR), jnp.int32),
            jax.ShapeDtypeStruct((n, 128), jnp.float32),
        ],
    )(x2d, wr_pad)


def _dense_ffn_body(mask_ref, x_ref, w1_ref, w2_ref, o_ref):
    e = pl.program_id(1)
    j = pl.program_id(2)
    x = x_ref[...]
    h = jax.lax.dot_general(
        x, w1_ref[0], (((1,), (1,)), ((), ())),
        preferred_element_type=jnp.float32)
    h = h * (1.0 / (1.0 + jnp.exp(-h)))
    c = jax.lax.dot_general(
        h, w2_ref[0], (((1,), (1,)), ((), ())),
        preferred_element_type=jnp.float32)
    m = mask_ref[...]
    col = jax.lax.broadcasted_iota(jnp.int32, m.shape, 1)
    msel = jnp.max(jnp.where(col == e, m, 0.0), axis=1, keepdims=True)
    c = c * msel

    @pl.when((e == 0) & (j == 0))
    def _init():
        o_ref[...] = c

    @pl.when((e > 0) | (j > 0))
    def _acc():
        o_ref[...] += c


def _dense_ffn(maskp, x2d, W1, W2):
    n = x2d.shape[0]
    g = n // _T
    nf = D_FF // _FB
    return pl.pallas_call(
        _dense_ffn_body,
        grid=(g, NUM_EXPERTS, nf),
        in_specs=[
            pl.BlockSpec((_T, 128), lambda i, e, j: (i, 0)),
            pl.BlockSpec((_T, D_MODEL), lambda i, e, j: (i, 0)),
            pl.BlockSpec((1, _FB, D_MODEL), lambda i, e, j: (e, j, 0)),
            pl.BlockSpec((1, D_MODEL, _FB), lambda i, e, j: (e, 0, j)),
        ],
        out_specs=pl.BlockSpec((_T, D_MODEL), lambda i, e, j: (i, 0)),
        out_shape=jax.ShapeDtypeStruct((n, D_MODEL), jnp.float32),
    )(maskp, x2d, W1, W2)


def kernel(hidden_states, W_router, W1, W2):
    b, s, d = hidden_states.shape
    n = b * s
    x2d = hidden_states.reshape(n, d)
    wr_pad = jnp.zeros((128, d), jnp.float32).at[:NUM_EXPERTS].set(W_router)
    e1, e2, maskp = _route(x2d, wr_pad)
    out = _dense_ffn(maskp, x2d, W1, W2)
    return out.reshape(b, s, d)


# speedup vs baseline: 1.2607x; 1.2607x over previous
import functools

import jax
import jax.numpy as jnp
from jax import lax
from jax.experimental import pallas as pl
from jax.experimental.pallas import tpu as pltpu
from jax.experimental.pallas import tpu_sc as plsc

D_MODEL = 1024
D_FF = 4096
NUM_EXPERTS = 8

N_TOK = 4096
P = 2 * N_TOK
NW = 32
CH = P // NW
T = 256
R = 10240
NB = R // T
NBP = 48
RW = R // NW
TT = N_TOK // NW
_TR = 1024
_FB = 512
_NF = D_FF // _FB


def _router_body(x_ref, wr_ref, e1_ref, e2_ref):
    x = x_ref[...]
    logits = lax.dot_general(
        x, wr_ref[...], (((1,), (1,)), ((), ())),
        preferred_element_type=jnp.float32)
    col = lax.broadcasted_iota(jnp.int32, logits.shape, 1)
    neg = jnp.float32(-1e30)
    logits = jnp.where(col < NUM_EXPERTS, logits, neg)
    m1 = jnp.max(logits, axis=1, keepdims=True)
    e1 = jnp.min(jnp.where(logits == m1, col, 127), axis=1)
    l2 = jnp.where(col == e1[:, None], neg, logits)
    m2 = jnp.max(l2, axis=1, keepdims=True)
    e2 = jnp.min(jnp.where(l2 == m2, col, 127), axis=1)
    e1_ref[...] = e1.reshape(1, 1, _TR)
    e2_ref[...] = e2.reshape(1, 1, _TR)


def _route(x2d, wr_pad):
    n = x2d.shape[0]
    g = n // _TR
    espec = pl.BlockSpec((1, 1, _TR), lambda i: (i, 0, 0))
    return pl.pallas_call(
        _router_body,
        grid=(g,),
        in_specs=[
            pl.BlockSpec((_TR, D_MODEL), lambda i: (i, 0)),
            pl.BlockSpec((128, D_MODEL), lambda i: (0, 0)),
        ],
        out_specs=[espec, espec],
        out_shape=[
            jax.ShapeDtypeStruct((g, 1, _TR), jnp.int32),
            jax.ShapeDtypeStruct((g, 1, _TR), jnp.int32),
        ],
    )(x2d, wr_pad)


def _meta_body(ids_hbm, pos_hbm, src_hbm, be_hbm,
               ids_all, hist, base_v, pos_my, pos2, tok2, be_v, sem):
    wid = lax.axis_index("s") * 2 + lax.axis_index("c")
    lane = lax.broadcasted_iota(jnp.int32, (16,), 0)
    ones = jnp.ones((16,), jnp.int32)
    zero16 = jnp.zeros((16,), jnp.int32)

    pltpu.sync_copy(ids_hbm, ids_all)
    for cc in range(NW):
        hist[pl.ds(cc * 16, 16)] = zero16

    def cnt_body(i, carry):
        v = ids_all[pl.ds(i * 16, 16)]
        row = lax.shift_right_logical(i, 4)
        rowv = jnp.full((16,), row * 16, jnp.int32)
        plsc.addupdate_scatter(hist, [rowv + v], ones)
        return carry

    lax.fori_loop(0, P // 16, cnt_body, 0)

    total = zero16
    pre = zero16
    widv = jnp.full((16,), wid, jnp.int32)
    for cc in range(NW):
        cnt_c = hist[pl.ds(cc * 16, 16)]
        total = total + cnt_c
        pred = jnp.full((16,), cc, jnp.int32) < widv
        pre = pre + jnp.where(pred, cnt_c, zero16)

    pc = (total + (T - 1)) & (~(T - 1))
    gs = plsc.cumsum(pc) - pc
    base_v[...] = gs + pre

    for k in range(16):
        off = wid * CH + k * 16
        v = ids_all[pl.ds(off, 16)]
        bg = plsc.load_gather(base_v, [v])
        rank = zero16
        for e in range(NUM_EXPERTS):
            m = v == e
            r = plsc.cumsum(m.astype(jnp.int32))
            rank = jnp.where(m, r - 1, rank)
        posv = bg + rank
        plsc.addupdate_scatter(base_v, [v], ones)
        pos_my[pl.ds(k * 16, 16)] = posv
        pos2[k // 8, pl.ds((k % 8) * 16, 16)] = posv
        tokv = (jnp.full((16,), off, jnp.int32) + lane) & (N_TOK - 1)
        tok2[k // 8, pl.ds((k % 8) * 16, 16)] = tokv

    pltpu.sync_copy(pos_my, pos_hbm.at[pl.ds(wid * CH, CH)])
    for jj in range(2):
        pltpu.async_copy(tok2.at[jj], src_hbm.at[pos2.at[jj]], sem).wait()

    @pl.when(wid == 0)
    def _blocks():
        for cc in range(NBP // 16):
            bi = (jnp.full((16,), cc * 16, jnp.int32) + lane) * T
            bev = zero16
            for e in range(1, NUM_EXPERTS):
                ge = gs[e]
                bev = bev + (bi >= ge).astype(jnp.int32)
            be_v[pl.ds(cc * 16, 16)] = bev
        pltpu.sync_copy(be_v, be_hbm)


def _meta(ids):
    mesh = plsc.VectorSubcoreMesh(core_axis_name="c", subcore_axis_name="s")
    f = pl.kernel(
        _meta_body,
        out_type=[
            jax.ShapeDtypeStruct((P,), jnp.int32),
            jax.ShapeDtypeStruct((R,), jnp.int32),
            jax.ShapeDtypeStruct((NBP,), jnp.int32),
        ],
        mesh=mesh,
        compiler_params=pltpu.CompilerParams(needs_layout_passes=False),
        scratch_types=[
            pltpu.VMEM((P,), jnp.int32),
            pltpu.VMEM((NW * 16,), jnp.int32),
            pltpu.VMEM((16,), jnp.int32),
            pltpu.VMEM((CH,), jnp.int32),
            pltpu.VMEM((2, 128), jnp.int32),
            pltpu.VMEM((2, 128), jnp.int32),
            pltpu.VMEM((NBP,), jnp.int32),
            pltpu.SemaphoreType.DMA,
        ],
    )
    return f(ids)


def _gather_body(src_hbm, x_hbm, xg_hbm, idx_v, rows_v, sem):
    wid = lax.axis_index("s") * 2 + lax.axis_index("c")
    for j in range(5):
        pltpu.sync_copy(src_hbm.at[pl.ds(wid * RW + j * 64, 64)], idx_v.at[j])
    for j in range(5):
        for k in range(4):
            vv = idx_v[j, pl.ds(k * 16, 16)]
            idx_v[j, pl.ds(k * 16, 16)] = jnp.minimum(
                jnp.maximum(vv, 0), N_TOK - 1)
    for j in range(5):
        pltpu.async_copy(x_hbm.at[idx_v.at[j]], rows_v, sem).wait()
        pltpu.sync_copy(rows_v, xg_hbm.at[pl.ds(wid * RW + j * 64, 64)])


def _gather(src, x2d):
    mesh = plsc.VectorSubcoreMesh(core_axis_name="c", subcore_axis_name="s")
    f = pl.kernel(
        _gather_body,
        out_type=[jax.ShapeDtypeStruct((R, D_MODEL), jnp.float32)],
        mesh=mesh,
        scratch_types=[
            pltpu.VMEM((5, 64), jnp.int32),
            pltpu.VMEM((64, D_MODEL), jnp.float32),
            pltpu.SemaphoreType.DMA,
        ],
    )
    return f(src, x2d)[0]


def _ffn_body(be_ref, x_ref, w1_ref, w2_ref, o_ref):
    j = pl.program_id(1)
    x = x_ref[...]
    h = lax.dot_general(
        x, w1_ref[0], (((1,), (1,)), ((), ())),
        preferred_element_type=jnp.float32)
    h = h * (1.0 / (1.0 + jnp.exp(-h)))
    c = lax.dot_general(
        h, w2_ref[0], (((1,), (1,)), ((), ())),
        preferred_element_type=jnp.float32)

    @pl.when(j == 0)
    def _init():
        o_ref[...] = c

    @pl.when(j > 0)
    def _acc():
        o_ref[...] += c


def _ffn(be, xg, W1, W2):
    grid_spec = pltpu.PrefetchScalarGridSpec(
        num_scalar_prefetch=1,
        grid=(NB, _NF),
        in_specs=[
            pl.BlockSpec((T, D_MODEL), lambda g, j, be: (g, 0)),
            pl.BlockSpec((1, _FB, D_MODEL), lambda g, j, be: (be[g], j, 0)),
            pl.BlockSpec((1, D_MODEL, _FB), lambda g, j, be: (be[g], 0, j)),
        ],
        out_specs=pl.BlockSpec((T, D_MODEL), lambda g, j, be: (g, 0)),
    )
    return pl.pallas_call(
        _ffn_body,
        grid_spec=grid_spec,
        out_shape=jax.ShapeDtypeStruct((R, D_MODEL), jnp.float32),
    )(be, xg, W1, W2)


def _combine_body(pos_hbm, buf_hbm, out_hbm, i1_v, i2_v, r1_v, r2_v, o_v, sem):
    wid = lax.axis_index("s") * 2 + lax.axis_index("c")
    for j in range(4):
        b0 = wid * TT + j * 32
        pltpu.sync_copy(pos_hbm.at[pl.ds(b0, 32)], i1_v)
        pltpu.sync_copy(pos_hbm.at[pl.ds(N_TOK + b0, 32)], i2_v)
        pltpu.async_copy(buf_hbm.at[i1_v], r1_v, sem).wait()
        pltpu.async_copy(buf_hbm.at[i2_v], r2_v, sem).wait()

        def add_body(t, carry):
            for l in range(D_MODEL // 16):
                o_v[t, pl.ds(l * 16, 16)] = (
                    r1_v[t, pl.ds(l * 16, 16)] + r2_v[t, pl.ds(l * 16, 16)])
            return carry

        lax.fori_loop(0, 32, add_body, 0)
        pltpu.sync_copy(o_v, out_hbm.at[pl.ds(b0, 32)])


def _combine(pos, buf):
    mesh = plsc.VectorSubcoreMesh(core_axis_name="c", subcore_axis_name="s")
    f = pl.kernel(
        _combine_body,
        out_type=[jax.ShapeDtypeStruct((N_TOK, D_MODEL), jnp.float32)],
        mesh=mesh,
        scratch_types=[
            pltpu.VMEM((32,), jnp.int32),
            pltpu.VMEM((32,), jnp.int32),
            pltpu.VMEM((32, D_MODEL), jnp.float32),
            pltpu.VMEM((32, D_MODEL), jnp.float32),
            pltpu.VMEM((32, D_MODEL), jnp.float32),
            pltpu.SemaphoreType.DMA,
        ],
    )
    return f(pos, buf)[0]


def kernel(hidden_states, W_router, W1, W2):
    b, s, d = hidden_states.shape
    n = b * s
    x2d = hidden_states.reshape(n, d)
    wr_pad = jnp.zeros((128, d), jnp.float32).at[:NUM_EXPERTS].set(W_router)
    e1, e2 = _route(x2d, wr_pad)
    ids = jnp.concatenate([e1.reshape(-1), e2.reshape(-1)])
    pos, src, be = _meta(ids)
    xg = _gather(src, x2d)
    buf = _ffn(be, xg, W1, W2)
    out2d = _combine(pos, buf)
    return out2d.reshape(b, s, d)
